# baseline (speedup 1.0000x reference)
import functools

import jax
import jax.numpy as jnp
import numpy as np
from jax.experimental import pallas as pl
from jax.experimental.pallas import tpu as pltpu

N, NFEAT, NHID, NCLASS = 8192, 128, 32, 16

_ROT_A = (13, 15, 26, 6)
_ROT_B = (17, 29, 16, 24)
_INJ = ((1, 2), (2, 0), (0, 1), (1, 2), (2, 0))


def _rotl(x, r):
    return (x * jnp.uint32(1 << r)) | (x >> jnp.uint32(32 - r))


def _threefry_keep(x1, seed):
    ks = (np.uint32(0), np.uint32(seed), np.uint32(seed ^ 0x1BD11BDA))
    first = True
    x0 = None
    for g, (ia, ib) in enumerate(_INJ):
        for r in (_ROT_A if g % 2 == 0 else _ROT_B):
            if first:
                x0 = x1
                first = False
            else:
                x0 = x0 + x1
            x1 = _rotl(x1, r) ^ x0
        if ks[ia]:
            x0 = x0 + jnp.uint32(ks[ia])
        x1 = x1 + jnp.uint32(np.uint32(ks[ib]) + np.uint32(g + 1))
    bits = jax.lax.bitcast_convert_type(x0 ^ x1, jnp.int32)
    return bits >= 0


def _pass_kernel(adj_ref, s_ref, b_ref, out_ref, msk_ref, *, seed, br, cr,
                 last_act):
    r = pl.program_id(0)

    base = (r * (br * N) + seed).astype(jnp.uint32)
    row = jax.lax.broadcasted_iota(jnp.uint32, (cr, N), 0)
    col = jax.lax.broadcasted_iota(jnp.uint32, (cr, N), 1)
    idx0 = base + row * jnp.uint32(N) + col

    def body(i, _):
        idx = idx0 + (i * (cr * N)).astype(jnp.uint32)
        keep = _threefry_keep(idx, seed)
        sl = pl.ds(i * cr, cr)
        msk_ref[sl, :] = jnp.where(keep, adj_ref[sl, :],
                                   jnp.float32(0.0)).astype(jnp.bfloat16)
        return 0

    jax.lax.fori_loop(0, br // cr, body, 0, unroll=2)

    y = jnp.dot(msk_ref[...], s_ref[...].astype(jnp.bfloat16),
                preferred_element_type=jnp.float32) + b_ref[...]
    if last_act == "relu":
        y = jnp.maximum(y, jnp.float32(0.0))
    else:
        m = jnp.max(y, axis=1, keepdims=True)
        sh = y - m
        lse = jnp.log(jnp.sum(jnp.exp(sh), axis=1, keepdims=True))
        y = sh - lse
    out_ref[...] = y


def _masked_spmm(adj, s, b, *, seed, last_act, br=512, cr=16):
    w = s.shape[1]
    return pl.pallas_call(
        functools.partial(_pass_kernel, seed=seed, br=br, cr=cr,
                          last_act=last_act),
        grid=(N // br,),
        in_specs=[
            pl.BlockSpec((br, N), lambda r: (r, 0)),
            pl.BlockSpec((N, w), lambda r: (0, 0)),
            pl.BlockSpec((1, w), lambda r: (0, 0)),
        ],
        out_specs=pl.BlockSpec((br, w), lambda r: (r, 0)),
        out_shape=jax.ShapeDtypeStruct((N, w), jnp.float32),
        scratch_shapes=[pltpu.VMEM((br, N), jnp.bfloat16)],
        compiler_params=pltpu.CompilerParams(
            dimension_semantics=("parallel",)),
    )(adj, s, b.reshape(1, w))


def _mm_kernel(a_ref, w_ref, out_ref):
    out_ref[...] = jnp.dot(a_ref[...], w_ref[...],
                           preferred_element_type=jnp.float32)


def _small_mm(a, w, br=1024):
    k = a.shape[1]
    n = w.shape[1]
    return pl.pallas_call(
        _mm_kernel,
        grid=(N // br,),
        in_specs=[
            pl.BlockSpec((br, k), lambda r: (r, 0)),
            pl.BlockSpec((k, n), lambda r: (0, 0)),
        ],
        out_specs=pl.BlockSpec((br, n), lambda r: (r, 0)),
        out_shape=jax.ShapeDtypeStruct((N, n), jnp.float32),
    )(a, w)


def kernel(x, adj, W1, b1, W2, b2):
    s1 = _small_mm(x, W1)
    h = _masked_spmm(adj, s1, b1, seed=1, last_act="relu")
    s2 = _small_mm(h, W2)
    out = _masked_spmm(adj, s2, b2, seed=2, last_act="log_softmax")
    return out


# speedup vs baseline: 1.0076x; 1.0076x over previous
import functools

import jax
import jax.numpy as jnp
import numpy as np
from jax.experimental import pallas as pl
from jax.experimental.pallas import tpu as pltpu

N, NFEAT, NHID, NCLASS = 8192, 128, 32, 16

_ROT_A = (13, 15, 26, 6)
_ROT_B = (17, 29, 16, 24)
_INJ = ((1, 2), (2, 0), (0, 1), (1, 2), (2, 0))


def _rotl(x, r):
    return (x * jnp.uint32(1 << r)) | (x >> jnp.uint32(32 - r))


def _threefry_keep(x1, seed):
    ks = (np.uint32(0), np.uint32(seed), np.uint32(seed ^ 0x1BD11BDA))
    first = True
    x0 = None
    for g, (ia, ib) in enumerate(_INJ):
        for r in (_ROT_A if g % 2 == 0 else _ROT_B):
            if first:
                x0 = x1
                first = False
            else:
                x0 = x0 + x1
            x1 = _rotl(x1, r) ^ x0
        if ks[ia]:
            x0 = x0 + jnp.uint32(ks[ia])
        x1 = x1 + jnp.uint32(np.uint32(ks[ib]) + np.uint32(g + 1))
    bits = jax.lax.bitcast_convert_type(x0 ^ x1, jnp.int32)
    return bits >= 0


def _pass_kernel(adj_ref, s_ref, b_ref, out_ref, msk_ref, *, seed, br, cr,
                 last_act):
    r = pl.program_id(0)

    base = (r * (br * N) + seed).astype(jnp.uint32)
    row = jax.lax.broadcasted_iota(jnp.uint32, (cr, N), 0)
    col = jax.lax.broadcasted_iota(jnp.uint32, (cr, N), 1)
    idx0 = base + row * jnp.uint32(N) + col

    def body(i, _):
        idx = idx0 + (i * (cr * N)).astype(jnp.uint32)
        keep = _threefry_keep(idx, seed)
        sl = pl.ds(i * cr, cr)
        msk_ref[sl, :] = jnp.where(keep, adj_ref[sl, :], jnp.float32(0.0))
        return 0

    jax.lax.fori_loop(0, br // cr, body, 0, unroll=4)

    y = jnp.dot(msk_ref[...], s_ref[...],
                preferred_element_type=jnp.float32) + b_ref[...]
    if last_act == "relu":
        y = jnp.maximum(y, jnp.float32(0.0))
    else:
        m = jnp.max(y, axis=1, keepdims=True)
        sh = y - m
        lse = jnp.log(jnp.sum(jnp.exp(sh), axis=1, keepdims=True))
        y = sh - lse
    out_ref[...] = y


def _masked_spmm(adj, s, b, *, seed, last_act, br=512, cr=16):
    w = s.shape[1]
    return pl.pallas_call(
        functools.partial(_pass_kernel, seed=seed, br=br, cr=cr,
                          last_act=last_act),
        grid=(N // br,),
        in_specs=[
            pl.BlockSpec((br, N), lambda r: (r, 0)),
            pl.BlockSpec((N, w), lambda r: (0, 0)),
            pl.BlockSpec((1, w), lambda r: (0, 0)),
        ],
        out_specs=pl.BlockSpec((br, w), lambda r: (r, 0)),
        out_shape=jax.ShapeDtypeStruct((N, w), jnp.float32),
        scratch_shapes=[pltpu.VMEM((br, N), jnp.float32)],
        compiler_params=pltpu.CompilerParams(
            dimension_semantics=("parallel",)),
    )(adj, s, b.reshape(1, w))


def _mm_kernel(a_ref, w_ref, out_ref):
    out_ref[...] = jnp.dot(a_ref[...], w_ref[...],
                           preferred_element_type=jnp.float32)


def _small_mm(a, w, br=1024):
    k = a.shape[1]
    n = w.shape[1]
    return pl.pallas_call(
        _mm_kernel,
        grid=(N // br,),
        in_specs=[
            pl.BlockSpec((br, k), lambda r: (r, 0)),
            pl.BlockSpec((k, n), lambda r: (0, 0)),
        ],
        out_specs=pl.BlockSpec((br, n), lambda r: (r, 0)),
        out_shape=jax.ShapeDtypeStruct((N, n), jnp.float32),
    )(a, w)


def kernel(x, adj, W1, b1, W2, b2):
    s1 = _small_mm(x, W1)
    h = _masked_spmm(adj, s1, b1, seed=1, last_act="relu")
    s2 = _small_mm(h, W2)
    out = _masked_spmm(adj, s2, b2, seed=2, last_act="log_softmax")
    return out


# speedup vs baseline: 1.0135x; 1.0058x over previous
import functools

import jax
import jax.numpy as jnp
import numpy as np
from jax.experimental import pallas as pl
from jax.experimental.pallas import tpu as pltpu

N, NFEAT, NHID, NCLASS = 8192, 128, 32, 16

_ROT_A = (13, 15, 26, 6)
_ROT_B = (17, 29, 16, 24)
_INJ = ((1, 2), (2, 0), (0, 1), (1, 2), (2, 0))


def _rotl(x, r):
    return (x * jnp.uint32(1 << r)) | (x >> jnp.uint32(32 - r))


def _threefry_keep(x1, seed):
    ks = (np.uint32(0), np.uint32(seed), np.uint32(seed ^ 0x1BD11BDA))
    first = True
    x0 = None
    for g, (ia, ib) in enumerate(_INJ):
        for r in (_ROT_A if g % 2 == 0 else _ROT_B):
            if first:
                x0 = x1
                first = False
            else:
                x0 = x0 + x1
            x1 = _rotl(x1, r) ^ x0
        if ks[ia]:
            x0 = x0 + jnp.uint32(ks[ia])
        x1 = x1 + jnp.uint32(np.uint32(ks[ib]) + np.uint32(g + 1))
    bits = jax.lax.bitcast_convert_type(x0 ^ x1, jnp.int32)
    return bits >= 0


def _pass1_kernel(adj_ref, x_ref, w1_ref, b_ref, w2_ref, out_ref, msk_ref,
                  s_ref, *, br, cr):
    seed = 1
    r = pl.program_id(0)

    @pl.when(r == 0)
    def _build_s1():
        s_ref[...] = jnp.dot(x_ref[...], w1_ref[...],
                             preferred_element_type=jnp.float32)

    base = (r * (br * N) + seed).astype(jnp.uint32)
    row = jax.lax.broadcasted_iota(jnp.uint32, (cr, N), 0)
    col = jax.lax.broadcasted_iota(jnp.uint32, (cr, N), 1)
    idx0 = base + row * jnp.uint32(N) + col

    def body(i, _):
        idx = idx0 + (i * (cr * N)).astype(jnp.uint32)
        keep = _threefry_keep(idx, seed)
        sl = pl.ds(i * cr, cr)
        msk_ref[sl, :] = jnp.where(keep, adj_ref[sl, :], jnp.float32(0.0))
        return 0

    jax.lax.fori_loop(0, br // cr, body, 0, unroll=4)

    h = jnp.maximum(jnp.dot(msk_ref[...], s_ref[...],
                            preferred_element_type=jnp.float32) + b_ref[...],
                    jnp.float32(0.0))
    out_ref[...] = jnp.dot(h, w2_ref[...], preferred_element_type=jnp.float32)


def _pass1(adj, x, W1, b1, W2, *, br=512, cr=16):
    return pl.pallas_call(
        functools.partial(_pass1_kernel, br=br, cr=cr),
        grid=(N // br,),
        in_specs=[
            pl.BlockSpec((br, N), lambda r: (r, 0)),
            pl.BlockSpec((N, NFEAT), lambda r: (0, 0)),
            pl.BlockSpec((NFEAT, NHID), lambda r: (0, 0)),
            pl.BlockSpec((1, NHID), lambda r: (0, 0)),
            pl.BlockSpec((NHID, NCLASS), lambda r: (0, 0)),
        ],
        out_specs=pl.BlockSpec((br, NCLASS), lambda r: (r, 0)),
        out_shape=jax.ShapeDtypeStruct((N, NCLASS), jnp.float32),
        scratch_shapes=[pltpu.VMEM((br, N), jnp.float32),
                        pltpu.VMEM((N, NHID), jnp.float32)],
        compiler_params=pltpu.CompilerParams(
            dimension_semantics=("arbitrary",)),
    )(adj, x, W1, b1.reshape(1, NHID), W2)


def _pass_kernel(adj_ref, s_ref, b_ref, out_ref, msk_ref, *, seed, br, cr,
                 last_act):
    r = pl.program_id(0)

    base = (r * (br * N) + seed).astype(jnp.uint32)
    row = jax.lax.broadcasted_iota(jnp.uint32, (cr, N), 0)
    col = jax.lax.broadcasted_iota(jnp.uint32, (cr, N), 1)
    idx0 = base + row * jnp.uint32(N) + col

    def body(i, _):
        idx = idx0 + (i * (cr * N)).astype(jnp.uint32)
        keep = _threefry_keep(idx, seed)
        sl = pl.ds(i * cr, cr)
        msk_ref[sl, :] = jnp.where(keep, adj_ref[sl, :], jnp.float32(0.0))
        return 0

    jax.lax.fori_loop(0, br // cr, body, 0, unroll=4)

    y = jnp.dot(msk_ref[...], s_ref[...],
                preferred_element_type=jnp.float32) + b_ref[...]
    if last_act == "relu":
        y = jnp.maximum(y, jnp.float32(0.0))
    else:
        m = jnp.max(y, axis=1, keepdims=True)
        sh = y - m
        lse = jnp.log(jnp.sum(jnp.exp(sh), axis=1, keepdims=True))
        y = sh - lse
    out_ref[...] = y


def _masked_spmm(adj, s, b, *, seed, last_act, br=512, cr=16):
    w = s.shape[1]
    return pl.pallas_call(
        functools.partial(_pass_kernel, seed=seed, br=br, cr=cr,
                          last_act=last_act),
        grid=(N // br,),
        in_specs=[
            pl.BlockSpec((br, N), lambda r: (r, 0)),
            pl.BlockSpec((N, w), lambda r: (0, 0)),
            pl.BlockSpec((1, w), lambda r: (0, 0)),
        ],
        out_specs=pl.BlockSpec((br, w), lambda r: (r, 0)),
        out_shape=jax.ShapeDtypeStruct((N, w), jnp.float32),
        scratch_shapes=[pltpu.VMEM((br, N), jnp.float32)],
        compiler_params=pltpu.CompilerParams(
            dimension_semantics=("parallel",)),
    )(adj, s, b.reshape(1, w))


def kernel(x, adj, W1, b1, W2, b2):
    s2 = _pass1(adj, x, W1, b1, W2)
    out = _masked_spmm(adj, s2, b2, seed=2, last_act="log_softmax")
    return out
